# baseline (speedup 1.0000x reference)
import jax
import jax.numpy as jnp
from jax.experimental import pallas as pl
from jax.experimental.pallas import tpu as pltpu

B, C, H, W = 16, 1024, 32, 32
P = H * W
D = 256


def _dvae_kernel(shift_ref, probs_ref, emb_ref, x_ref, tok_ref):
    p = probs_ref[0]
    iota_c = jax.lax.broadcasted_iota(jnp.int32, (C, P), 0)
    maxv = jnp.max(p, axis=0, keepdims=True)
    idx = jnp.min(jnp.where(p == maxv, iota_c, C), axis=0, keepdims=True)
    tok = idx + shift_ref[0, 0]
    tok_ref[0] = tok
    onehot = (iota_c == tok).astype(jnp.float32)
    x_ref[0] = jax.lax.dot_general(
        emb_ref[...], onehot,
        dimension_numbers=(((0,), (0,)), ((), ())),
        preferred_element_type=jnp.float32,
    )


def kernel(probs, tokens_shift, embedding_weight):
    probs2 = probs.reshape(B, C, P)
    shift = jnp.asarray(tokens_shift, jnp.int32).reshape(1, 1)
    x, tok = pl.pallas_call(
        _dvae_kernel,
        grid=(B,),
        in_specs=[
            pl.BlockSpec(memory_space=pltpu.SMEM),
            pl.BlockSpec((1, C, P), lambda b: (b, 0, 0)),
            pl.BlockSpec((C, D), lambda b: (0, 0)),
        ],
        out_specs=[
            pl.BlockSpec((1, D, P), lambda b: (b, 0, 0)),
            pl.BlockSpec((1, 1, P), lambda b: (b, 0, 0)),
        ],
        out_shape=[
            jax.ShapeDtypeStruct((B, D, P), jnp.float32),
            jax.ShapeDtypeStruct((B, 1, P), jnp.int32),
        ],
    )(shift, probs2, embedding_weight)
    return (x.reshape(B, D, H, W), tok.reshape(B, H, W))
